# Initial kernel scaffold; baseline (speedup 1.0000x reference)
import jax
import jax.numpy as jnp
from jax.experimental import pallas as pl


def kernel(distances, table):
    raise NotImplementedError("write your pallas kernel here")


# speedup vs baseline: 8.7277x; 8.7277x over previous
import functools

import jax
import jax.numpy as jnp
from jax import lax
from jax.experimental import pallas as pl
from jax.experimental.pallas import tpu as pltpu
from jax.experimental.pallas import tpu_sc as plsc

_H = 16
_V = 32
_N = 1024
_TOTAL = _N * _N
_NC = 2
_NS = 16
_LANES = 16
_NW = _NC * _NS
_PER_W = _TOTAL // _NW
_CHUNK = 2048
_NSTEP = _PER_W // _CHUNK


def _gdb_body(d_hbm, tab_hbm, out_hbm, tab_v, d_v, o_v):
    wid = lax.axis_index("s") * _NC + lax.axis_index("c")
    base_w = wid * _PER_W

    pltpu.sync_copy(tab_hbm, tab_v)

    def step(g, carry):
        base = pl.multiple_of(base_w + g * _CHUNK, _CHUNK)
        pltpu.sync_copy(d_hbm.at[pl.ds(base, _CHUNK)], d_v)

        def slice_body(s, c):
            off = s * _LANES
            d16 = d_v[pl.ds(off, _LANES)] * _H
            for h in range(_H):
                o_v[h, pl.ds(off, _LANES)] = plsc.load_gather(
                    tab_v, [d16 + h])
            return c

        lax.fori_loop(0, _CHUNK // _LANES, slice_body, 0, unroll=4)

        for h in range(_H):
            pltpu.sync_copy(o_v.at[h], out_hbm.at[h, pl.ds(base, _CHUNK)])
        return carry

    lax.fori_loop(0, _NSTEP, step, 0)


def kernel(distances, table):
    d_flat = distances.reshape(_TOTAL).astype(jnp.int32)
    tab_flat = table.reshape(_V * _H)

    mesh = plsc.VectorSubcoreMesh(
        core_axis_name="c", subcore_axis_name="s",
        num_cores=_NC, num_subcores=_NS)

    run = pl.kernel(
        _gdb_body,
        out_type=jax.ShapeDtypeStruct((_H, _TOTAL), jnp.float32),
        mesh=mesh,
        scratch_types=[
            pltpu.VMEM((_V * _H,), jnp.float32),
            pltpu.VMEM((_CHUNK,), jnp.int32),
            pltpu.VMEM((_H, _CHUNK), jnp.float32),
        ],
        compiler_params=pltpu.CompilerParams(needs_layout_passes=False),
    )
    out = run(d_flat, tab_flat)
    return out.reshape(_H, _N, _N)


# speedup vs baseline: 9.6046x; 1.1005x over previous
import jax
import jax.numpy as jnp
from jax import lax
from jax.experimental import pallas as pl
from jax.experimental.pallas import tpu as pltpu
from jax.experimental.pallas import tpu_sc as plsc

_H = 16
_V = 32
_N = 1024
_TOTAL = _N * _N
_NC = 2
_NS = 16
_LANES = 16
_NW = _NC * _NS
_PER_W = _TOTAL // _NW
_CHUNK = 2048
_NSTEP = _PER_W // _CHUNK


def _gdb_body(d_hbm, tab_hbm, out_hbm, tab_v, d_v, o_v,
              dsem0, dsem1, osem0, osem1):
    wid = lax.axis_index("s") * _NC + lax.axis_index("c")
    base_w = wid * _PER_W
    dsems = (dsem0, dsem1)
    osems = (osem0, osem1)

    pltpu.sync_copy(tab_hbm, tab_v)

    def start_d(g, b):
        base = pl.multiple_of(base_w + g * _CHUNK, _CHUNK)
        return pltpu.async_copy(
            d_hbm.at[pl.ds(base, _CHUNK)], d_v.at[b], dsems[b])

    def start_o(g, b):
        base = pl.multiple_of(base_w + g * _CHUNK, _CHUNK)
        return pltpu.async_copy(
            o_v.at[b], out_hbm.at[:, pl.ds(base, _CHUNK)], osems[b])

    def compute(b):
        def slice_body(s, c):
            off = s * _LANES
            d16 = d_v[b, pl.ds(off, _LANES)] * _H
            for h in range(_H):
                o_v[b, h, pl.ds(off, _LANES)] = plsc.load_gather(
                    tab_v, [d16 + h])
            return c
        lax.fori_loop(0, _CHUNK // _LANES, slice_body, 0, unroll=2)

    d_copies = {0: start_d(0, 0)}
    o_copies = {}
    for g in range(_NSTEP):
        b = g % 2
        if g + 1 < _NSTEP:
            d_copies[g + 1] = start_d(g + 1, 1 - b)
        d_copies[g].wait()
        if g >= 2:
            o_copies[g - 2].wait()
        compute(b)
        o_copies[g] = start_o(g, b)
    o_copies[_NSTEP - 2].wait()
    o_copies[_NSTEP - 1].wait()


def kernel(distances, table):
    d_flat = distances.reshape(_TOTAL).astype(jnp.int32)
    tab_flat = table.reshape(_V * _H)

    mesh = plsc.VectorSubcoreMesh(
        core_axis_name="c", subcore_axis_name="s",
        num_cores=_NC, num_subcores=_NS)

    run = pl.kernel(
        _gdb_body,
        out_type=jax.ShapeDtypeStruct((_H, _TOTAL), jnp.float32),
        mesh=mesh,
        scratch_types=[
            pltpu.VMEM((_V * _H,), jnp.float32),
            pltpu.VMEM((2, _CHUNK), jnp.int32),
            pltpu.VMEM((2, _H, _CHUNK), jnp.float32),
            pltpu.SemaphoreType.DMA,
            pltpu.SemaphoreType.DMA,
            pltpu.SemaphoreType.DMA,
            pltpu.SemaphoreType.DMA,
        ],
        compiler_params=pltpu.CompilerParams(needs_layout_passes=False),
    )
    out = run(d_flat, tab_flat)
    return out.reshape(_H, _N, _N)


# speedup vs baseline: 21.3047x; 2.2182x over previous
import jax
import jax.numpy as jnp
from jax import lax
from jax.experimental import pallas as pl
from jax.experimental.pallas import tpu as pltpu
from jax.experimental.pallas import tpu_sc as plsc

_H = 16
_V = 32
_N = 1024
_TOTAL = _N * _N
_NC = 2
_NS = 16
_LANES = 16
_NW = _NC * _NS
_ROWS_W = _N // _NW
_R = 2
_NSTEP = _ROWS_W // _R
_CHUNK = _R * _N


def _gdb_body(d_hbm, tabT_hbm, out_hbm, cols_v, d_v, o_v,
              dsem0, dsem1, osem0, osem1):
    wid = lax.axis_index("s") * _NC + lax.axis_index("c")
    row_w = wid * _ROWS_W
    dsems = (dsem0, dsem1)
    osems = (osem0, osem1)

    pltpu.sync_copy(tabT_hbm, cols_v)

    def start_d(g, b):
        r0 = row_w + g * _R
        return pltpu.async_copy(
            d_hbm.at[pl.ds(r0, _R), :], d_v.at[b], dsems[b])

    def start_o(g, b):
        r0 = row_w + g * _R
        return pltpu.async_copy(
            o_v.at[b], out_hbm.at[:, pl.ds(r0, _R), :], osems[b])

    def wait_d(b):
        pltpu.make_async_copy(
            d_hbm.at[pl.ds(0, _R), :], d_v.at[b], dsems[b]).wait()

    def wait_o(b):
        pltpu.make_async_copy(
            o_v.at[b], out_hbm.at[:, pl.ds(0, _R), :], osems[b]).wait()

    def compute(b):
        for r in range(_R):
            def slice_body(s, c, r=r):
                off = s * _LANES
                d = d_v[b, r, pl.ds(off, _LANES)]
                for h in range(_H):
                    o_v[b, h, r, pl.ds(off, _LANES)] = plsc.load_gather(
                        cols_v.at[h], [d])
                return c
            lax.fori_loop(0, _N // _LANES, slice_body, 0, unroll=2)

    start_d(0, 0)
    start_d(1, 1)

    def pair_body(g0, c):
        for b in range(2):
            g = 2 * g0 + b
            wait_d(b)

            @pl.when(g >= 2)
            def _():
                wait_o(b)

            compute(b)
            start_o(g, b)

            @pl.when(g + 2 < _NSTEP)
            def _():
                start_d(g + 2, b)
        return c

    lax.fori_loop(0, _NSTEP // 2, pair_body, 0)
    wait_o(0)
    wait_o(1)


def kernel(distances, table):
    d_2d = distances.astype(jnp.int32)
    tab_t = table.T.reshape(_H, _V)

    mesh = plsc.VectorSubcoreMesh(
        core_axis_name="c", subcore_axis_name="s",
        num_cores=_NC, num_subcores=_NS)

    run = pl.kernel(
        _gdb_body,
        out_type=jax.ShapeDtypeStruct((_H, _N, _N), jnp.float32),
        mesh=mesh,
        scratch_types=[
            pltpu.VMEM((_H, _V), jnp.float32),
            pltpu.VMEM((2, _R, _N), jnp.int32),
            pltpu.VMEM((2, _H, _R, _N), jnp.float32),
            pltpu.SemaphoreType.DMA,
            pltpu.SemaphoreType.DMA,
            pltpu.SemaphoreType.DMA,
            pltpu.SemaphoreType.DMA,
        ],
        compiler_params=pltpu.CompilerParams(needs_layout_passes=False),
    )
    return run(d_2d, tab_t)
